# baseline (speedup 1.0000x reference)
import functools
import math

import jax
import jax.numpy as jnp
from jax import lax
from jax.experimental import pallas as pl
from jax.experimental.pallas import tpu as pltpu
from jax.experimental.pallas import tpu_sc as plsc

D_MODEL = 128
LANES = 16
NUM_CORES = 2
NUM_SUBCORES = 16
CHUNK = 64
FAST_CHUNKS = 9
SLOW_CHUNKS = 7
FAST_CORE = 0
SCALE = math.sqrt(float(D_MODEL))


@functools.lru_cache(maxsize=None)
def _build(batch: int):
    assert batch // CHUNK == NUM_SUBCORES * (FAST_CHUNKS + SLOW_CHUNKS)
    fast_span = FAST_CHUNKS * CHUNK
    slow_span = SLOW_CHUNKS * CHUNK
    fast_total = NUM_SUBCORES * fast_span

    mesh = plsc.VectorSubcoreMesh(core_axis_name="c", subcore_axis_name="s",
                                  num_cores=NUM_CORES,
                                  num_subcores=NUM_SUBCORES)

    @functools.partial(
        pl.kernel,
        out_type=jax.ShapeDtypeStruct((batch, D_MODEL), jnp.float32),
        mesh=mesh,
        scratch_types=[
            pltpu.VMEM((fast_span,), jnp.int32),
            pltpu.VMEM((fast_span, D_MODEL), jnp.float32),
            pltpu.SemaphoreType.DMA,
            [pltpu.SemaphoreType.DMA] * FAST_CHUNKS,
            pltpu.SemaphoreType.DMA,
        ],
    )
    def emb_kernel(x_hbm, lut_hbm, out_hbm, idx_v, rows_v, isem, gsems, wsem):
        c = lax.axis_index("c")
        s = lax.axis_index("s")
        start = jnp.where(c == FAST_CORE, s * fast_span,
                          fast_total + s * slow_span)
        load_at = jnp.minimum(start, batch - fast_span)
        sub = start - load_at

        pltpu.async_copy(x_hbm.at[pl.ds(load_at, fast_span)], idx_v,
                         isem).wait()

        def gather(j):
            return pltpu.make_async_copy(
                lut_hbm.at[idx_v.at[pl.ds(sub + j * CHUNK, CHUNK)]],
                rows_v.at[pl.ds(j * CHUNK, CHUNK)], gsems[j])

        def write(j):
            return pltpu.make_async_copy(
                rows_v.at[pl.ds(j * CHUNK, CHUNK)],
                out_hbm.at[pl.ds(start + j * CHUNK, CHUNK)], wsem)

        def scale_rows(off):
            @plsc.parallel_loop(off, off + CHUNK, unroll=1)
            def _(r):
                for c8 in range(D_MODEL // LANES):
                    sl = rows_v[r, pl.ds(c8 * LANES, LANES)]
                    rows_v[r, pl.ds(c8 * LANES, LANES)] = sl * SCALE

        is_fast = jnp.where(c == FAST_CORE, 1, 0)
        extras = list(range(SLOW_CHUNKS, FAST_CHUNKS))

        for j in range(SLOW_CHUNKS):
            gather(j).start()

        @pl.loop(0, is_fast)
        def _(t):
            for j in extras:
                gather(j).start()
            for j in extras:
                gather(j).wait()
                scale_rows(j * CHUNK)
                write(j).start()
            for j in extras:
                write(j).wait()

        for j in range(SLOW_CHUNKS):
            gather(j).wait()
            scale_rows(j * CHUNK)
            write(j).start()

        for j in range(SLOW_CHUNKS):
            write(j).wait()

    return emb_kernel


def kernel(x, lut):
    b0, b1 = x.shape
    xf = jnp.ravel(x)
    if xf.dtype != jnp.int32:
        xf = xf.astype(jnp.int32)
    out = _build(b0 * b1)(xf, lut)
    return out.reshape(b0, b1, D_MODEL)


# speedup vs baseline: 1.0703x; 1.0703x over previous
import functools
import math

import jax
import jax.numpy as jnp
from jax import lax
from jax.experimental import pallas as pl
from jax.experimental.pallas import tpu as pltpu
from jax.experimental.pallas import tpu_sc as plsc

D_MODEL = 128
LANES = 16
NUM_CORES = 2
NUM_SUBCORES = 16
NUM_WORKERS = NUM_CORES * NUM_SUBCORES
CHUNK = 64
SCALE = math.sqrt(float(D_MODEL))


@functools.lru_cache(maxsize=None)
def _build(b0: int, b1: int):
    batch = b0 * b1
    assert batch % (NUM_WORKERS * CHUNK) == 0
    bpw = batch // NUM_WORKERS
    nchunk = bpw // CHUNK
    assert b1 % bpw == 0
    tiles_per_row = b1 // bpw

    mesh = plsc.VectorSubcoreMesh(core_axis_name="c", subcore_axis_name="s",
                                  num_cores=NUM_CORES,
                                  num_subcores=NUM_SUBCORES)

    @functools.partial(
        pl.kernel,
        out_type=jax.ShapeDtypeStruct((batch, D_MODEL), jnp.float32),
        mesh=mesh,
        scratch_types=[
            pltpu.VMEM((bpw,), jnp.int32),
            pltpu.VMEM((bpw, D_MODEL), jnp.float32),
            pltpu.SemaphoreType.DMA,
            [pltpu.SemaphoreType.DMA] * nchunk,
            pltpu.SemaphoreType.DMA,
        ],
    )
    def emb_kernel(x_hbm, lut_hbm, out_hbm, idx_v, rows_v, isem, gsems, wsem):
        wid = lax.axis_index("s") * NUM_CORES + lax.axis_index("c")
        base = wid * bpw
        row = wid // tiles_per_row
        col = (wid % tiles_per_row) * bpw

        pltpu.async_copy(x_hbm.at[row, pl.ds(col, bpw)], idx_v, isem).wait()

        gathers = [
            pltpu.async_copy(lut_hbm.at[idx_v.at[pl.ds(j * CHUNK, CHUNK)]],
                             rows_v.at[pl.ds(j * CHUNK, CHUNK)], gsems[j])
            for j in range(nchunk)
        ]

        for j in range(nchunk):
            off = j * CHUNK
            gathers[j].wait()

            @plsc.parallel_loop(off, off + CHUNK, unroll=1)
            def _(r):
                for c8 in range(D_MODEL // LANES):
                    sl = rows_v[r, pl.ds(c8 * LANES, LANES)]
                    rows_v[r, pl.ds(c8 * LANES, LANES)] = sl * SCALE

            pltpu.async_copy(rows_v.at[pl.ds(off, CHUNK)],
                             out_hbm.at[pl.ds(base + off, CHUNK)], wsem)

        pltpu.make_async_copy(rows_v, out_hbm.at[pl.ds(base, bpw)],
                              wsem).wait()

    return emb_kernel


def kernel(x, lut):
    b0, b1 = x.shape
    if x.dtype != jnp.int32:
        x = x.astype(jnp.int32)
    out = _build(b0, b1)(x, lut)
    return out.reshape(b0, b1, D_MODEL)


# speedup vs baseline: 1.0765x; 1.0058x over previous
import functools
import math

import jax
import jax.numpy as jnp
from jax import lax
from jax.experimental import pallas as pl
from jax.experimental.pallas import tpu as pltpu
from jax.experimental.pallas import tpu_sc as plsc

D_MODEL = 128
LANES = 16
NUM_CORES = 2
NUM_SUBCORES = 16
NUM_WORKERS = NUM_CORES * NUM_SUBCORES
CHUNK = 64
SCALE = math.sqrt(float(D_MODEL))


@functools.lru_cache(maxsize=None)
def _build(b0: int, b1: int):
    batch = b0 * b1
    assert batch % (NUM_WORKERS * CHUNK) == 0
    bpw = batch // NUM_WORKERS
    nchunk = bpw // CHUNK
    assert b1 % bpw == 0
    tiles_per_row = b1 // bpw

    mesh = plsc.VectorSubcoreMesh(core_axis_name="c", subcore_axis_name="s",
                                  num_cores=NUM_CORES,
                                  num_subcores=NUM_SUBCORES)

    @functools.partial(
        pl.kernel,
        out_type=jax.ShapeDtypeStruct((batch, D_MODEL), jnp.float32),
        mesh=mesh,
        scratch_types=[
            pltpu.VMEM((bpw,), jnp.int32),
            pltpu.VMEM((bpw, D_MODEL), jnp.float32),
            pltpu.SemaphoreType.DMA,
            [pltpu.SemaphoreType.DMA] * nchunk,
            pltpu.SemaphoreType.DMA,
        ],
    )
    def emb_kernel(x_hbm, lut_hbm, out_hbm, idx_v, rows_v, isem, gsems, wsem):
        wid = lax.axis_index("s") * NUM_CORES + lax.axis_index("c")
        base = wid * bpw
        row = wid // tiles_per_row
        col = (wid % tiles_per_row) * bpw

        pltpu.async_copy(x_hbm.at[row, pl.ds(col, bpw)], idx_v, isem).wait()

        gathers = [
            pltpu.async_copy(lut_hbm.at[idx_v.at[pl.ds(j * CHUNK, CHUNK)]],
                             rows_v.at[pl.ds(j * CHUNK, CHUNK)], gsems[j])
            for j in range(nchunk)
        ]

        for j in range(nchunk):
            off = j * CHUNK
            gathers[j].wait()

            @plsc.parallel_loop(off, off + CHUNK, unroll=2)
            def _(r):
                for c8 in range(D_MODEL // LANES):
                    sl = rows_v[r, pl.ds(c8 * LANES, LANES)]
                    rows_v[r, pl.ds(c8 * LANES, LANES)] = sl * SCALE

            pltpu.async_copy(rows_v.at[pl.ds(off, CHUNK)],
                             out_hbm.at[pl.ds(base + off, CHUNK)], wsem)

        pltpu.make_async_copy(rows_v, out_hbm.at[pl.ds(base, bpw)],
                              wsem).wait()

    return emb_kernel


def kernel(x, lut):
    b0, b1 = x.shape
    if x.dtype != jnp.int32:
        x = x.astype(jnp.int32)
    out = _build(b0, b1)(x, lut)
    return out.reshape(b0, b1, D_MODEL)
